# baseline (speedup 1.0000x reference)
import functools

import jax
import jax.numpy as jnp
from jax import lax
from jax.experimental import pallas as pl
from jax.experimental.pallas import tpu as pltpu
from jax.experimental.pallas import tpu_sc as plsc

_INFO = plsc.get_sparse_core_info()
_NC, _NS = _INFO.num_cores, _INFO.num_subcores
_NW = _NC * _NS
_CHUNK = 128


def _gather_body(b_per_w, n_chunks,
                 in_hbm, out_hbm, cidx_hbm, xidx_hbm,
                 cemb_hbm, xemb_hbm,
                 cidx_v, xidx_v, crows_v, xrows_v, csem, xsem):
    wid = lax.axis_index("s") * _NC + lax.axis_index("c")
    base = wid * b_per_w
    pltpu.sync_copy(cidx_hbm.at[pl.ds(base, b_per_w)], cidx_v)
    pltpu.sync_copy(xidx_hbm.at[pl.ds(base, b_per_w)], xidx_v)
    copies = []
    for j in range(n_chunks):
        sl = pl.ds(j * _CHUNK, _CHUNK)
        copies.append(
            pltpu.async_copy(in_hbm.at[cidx_v.at[sl]], crows_v.at[sl], csem))
        copies.append(
            pltpu.async_copy(out_hbm.at[xidx_v.at[sl]], xrows_v.at[sl], xsem))
    for cp in copies:
        cp.wait()
    pltpu.sync_copy(crows_v, cemb_hbm.at[pl.ds(base, b_per_w)])
    pltpu.sync_copy(xrows_v, xemb_hbm.at[pl.ds(base, b_per_w)])


def kernel(center_indices, context_indices, in_embeddings, out_embeddings):
    B, = center_indices.shape
    V, D = in_embeddings.shape
    assert B % (_NW * _CHUNK) == 0
    b_per_w = B // _NW
    n_chunks = b_per_w // _CHUNK

    mesh = plsc.VectorSubcoreMesh(core_axis_name="c", subcore_axis_name="s")
    emb = jax.ShapeDtypeStruct((B, D), jnp.float32)
    run = pl.kernel(
        functools.partial(_gather_body, b_per_w, n_chunks),
        mesh=mesh,
        out_type=(emb, emb),
        scratch_types=[
            pltpu.VMEM((b_per_w,), jnp.int32),
            pltpu.VMEM((b_per_w,), jnp.int32),
            pltpu.VMEM((b_per_w, D), jnp.float32),
            pltpu.VMEM((b_per_w, D), jnp.float32),
            pltpu.SemaphoreType.DMA,
            pltpu.SemaphoreType.DMA,
        ],
        compiler_params=pltpu.CompilerParams(use_tc_tiling_on_sc=False),
    )
    return run(in_embeddings, out_embeddings, center_indices, context_indices)


# speedup vs baseline: 1.0051x; 1.0051x over previous
import functools

import jax
import jax.numpy as jnp
from jax import lax
from jax.experimental import pallas as pl
from jax.experimental.pallas import tpu as pltpu
from jax.experimental.pallas import tpu_sc as plsc

_INFO = plsc.get_sparse_core_info()
_NC, _NS = _INFO.num_cores, _INFO.num_subcores
_NW = _NC * _NS
_CHUNK = 128


def _gather_body(b_per_w, n_chunks, table_hbm, idx_hbm, emb_hbm,
                 idx_v, rows_v, sem):
    wid = lax.axis_index("s") * _NC + lax.axis_index("c")
    base = wid * b_per_w
    pltpu.sync_copy(idx_hbm.at[pl.ds(base, b_per_w)], idx_v)
    copies = []
    for j in range(n_chunks):
        sl = pl.ds(j * _CHUNK, _CHUNK)
        copies.append(
            pltpu.async_copy(table_hbm.at[idx_v.at[sl]], rows_v.at[sl], sem))
    for cp in copies:
        cp.wait()
    pltpu.sync_copy(rows_v, emb_hbm.at[pl.ds(base, b_per_w)])


def _make_gather(B, V, D):
    b_per_w = B // _NW
    n_chunks = b_per_w // _CHUNK
    mesh = plsc.VectorSubcoreMesh(core_axis_name="c", subcore_axis_name="s")
    return pl.kernel(
        functools.partial(_gather_body, b_per_w, n_chunks),
        mesh=mesh,
        out_type=jax.ShapeDtypeStruct((B, D), jnp.float32),
        scratch_types=[
            pltpu.VMEM((b_per_w,), jnp.int32),
            pltpu.VMEM((b_per_w, D), jnp.float32),
            pltpu.SemaphoreType.DMA,
        ],
        compiler_params=pltpu.CompilerParams(use_tc_tiling_on_sc=False),
    )


def kernel(center_indices, context_indices, in_embeddings, out_embeddings):
    B, = center_indices.shape
    V, D = in_embeddings.shape
    assert B % (_NW * _CHUNK) == 0
    run = _make_gather(B, V, D)
    center_emb = run(in_embeddings, center_indices)
    context_emb = run(out_embeddings, context_indices)
    return (center_emb, context_emb)


# speedup vs baseline: 1.0745x; 1.0690x over previous
import functools

import jax
import jax.numpy as jnp
from jax import lax
from jax.experimental import pallas as pl
from jax.experimental.pallas import tpu as pltpu
from jax.experimental.pallas import tpu_sc as plsc

_INFO = plsc.get_sparse_core_info()
_NC, _NS = _INFO.num_cores, _INFO.num_subcores
_NW = _NC * _NS
_CHUNK = 128


def _body(b_per_w, n_chunks,
          in_hbm, out_hbm, cidx_hbm, xidx_hbm,
          cemb_hbm, xemb_hbm,
          cidx_v, xidx_v, rows_v, csem, xsem):
    wid = lax.axis_index("s") * _NC + lax.axis_index("c")
    base = wid * b_per_w
    pltpu.sync_copy(cidx_hbm.at[pl.ds(base, b_per_w)], cidx_v)
    pltpu.sync_copy(xidx_hbm.at[pl.ds(base, b_per_w)], xidx_v)
    for table, idx_v, emb, sem in (
        (in_hbm, cidx_v, cemb_hbm, csem),
        (out_hbm, xidx_v, xemb_hbm, xsem),
    ):
        copies = []
        for j in range(n_chunks):
            sl = pl.ds(j * _CHUNK, _CHUNK)
            copies.append(
                pltpu.async_copy(table.at[idx_v.at[sl]], rows_v.at[sl], sem))
        for cp in copies:
            cp.wait()
        pltpu.sync_copy(rows_v, emb.at[pl.ds(base, b_per_w)])


def kernel(center_indices, context_indices, in_embeddings, out_embeddings):
    B, = center_indices.shape
    V, D = in_embeddings.shape
    assert B % (_NW * _CHUNK) == 0 and D == 64
    b_per_w = B // _NW
    n_chunks = b_per_w // _CHUNK

    in_p = jnp.pad(in_embeddings, ((0, 0), (0, 128 - D)))
    out_p = jnp.pad(out_embeddings, ((0, 0), (0, 128 - D)))

    mesh = plsc.VectorSubcoreMesh(core_axis_name="c", subcore_axis_name="s")
    emb = jax.ShapeDtypeStruct((B, 128), jnp.float32)
    run = pl.kernel(
        functools.partial(_body, b_per_w, n_chunks),
        mesh=mesh,
        out_type=(emb, emb),
        scratch_types=[
            pltpu.VMEM((b_per_w,), jnp.int32),
            pltpu.VMEM((b_per_w,), jnp.int32),
            pltpu.VMEM((b_per_w, 128), jnp.float32),
            pltpu.SemaphoreType.DMA,
            pltpu.SemaphoreType.DMA,
        ],
    )
    c128, x128 = run(in_p, out_p, center_indices, context_indices)
    return (c128[:, :D], x128[:, :D])
